# baseline (speedup 1.0000x reference)
import functools

import jax
import jax.numpy as jnp
from jax import lax
from jax.experimental import pallas as pl
from jax.experimental.pallas import tpu as pltpu
from jax.experimental.pallas import tpu_sc as plsc

_K = 256
_ROWS = 128
_COLS = 8192
_L = 16
_CHUNKS = _COLS // _L
_NC = 2
_NS = 16
_NW = _NC * _NS
_RPW = _ROWS // _NW
_H = 4096


def kernel(x):
    mesh = plsc.VectorSubcoreMesh(core_axis_name="c", subcore_axis_name="s")

    @functools.partial(
        pl.kernel,
        mesh=mesh,
        out_type=jax.ShapeDtypeStruct((_ROWS, _COLS), jnp.float32),
        scratch_types=[
            pltpu.VMEM((_COLS,), jnp.float32),
            pltpu.VMEM((_COLS,), jnp.float32),
            pltpu.VMEM((_H,), jnp.int32),
            pltpu.VMEM((_COLS + _L,), jnp.int32),
            pltpu.VMEM((_COLS + _L,), jnp.int32),
            pltpu.VMEM((_COLS,), jnp.float32),
            pltpu.VMEM((_COLS,), jnp.float32),
            pltpu.VMEM((_K + _L,), jnp.int32),
            pltpu.VMEM((_K + _L,), jnp.int32),
            pltpu.VMEM((_K + _L,), jnp.int32),
            pltpu.SemaphoreType.DMA,
            pltpu.SemaphoreType.DMA,
            pltpu.SemaphoreType.DMA,
            pltpu.SemaphoreType.DMA,
        ],
        compiler_params=pltpu.CompilerParams(needs_layout_passes=False),
    )
    def wta(x_hbm, out_hbm, xva, xvb, hist, csk, cix, outva, outvb,
            ksk, kixa, kixb, sia, sib, soa, sob):
        zi16 = jnp.zeros((_L,), jnp.int32)
        zf16 = jnp.zeros((_L,), jnp.float32)
        ones16 = jnp.ones((_L,), jnp.int32)
        lanes = lax.iota(jnp.int32, _L)
        wid = lax.axis_index("s") * _NC + lax.axis_index("c")
        row0 = wid * _RPW
        xv2 = (xva, xvb)
        outv2 = (outva, outvb)
        kix2 = (kixa, kixb)
        si2 = (sia, sib)
        so2 = (soa, sob)

        h_in0 = pltpu.async_copy(x_hbm.at[row0], xva, sia)

        @plsc.parallel_loop(0, _H // _L, unroll=8)
        def _zero_hist(i):
            hist[pl.ds(i * _L, _L)] = zi16

        @plsc.parallel_loop(0, _CHUNKS, unroll=8)
        def _zero_out(i):
            outva[pl.ds(i * _L, _L)] = zf16
            outvb[pl.ds(i * _L, _L)] = zf16

        def sortkey(bits):
            return bits ^ ((bits >> 31) & jnp.int32(0x7FFFFFFF))

        def analyze(sv, sbase, stot, target):
            rc = lax.rev(sv, (0,))
            cum = stot + plsc.cumsum(rc)
            ge = cum >= target
            ge2 = (cum - rc) >= target
            popc = jnp.max(plsc.all_reduce_population_count(ge))
            h = sbase + popc - 1
            cab = stot + jnp.sum(jnp.where(ge, 0, rc))
            cnt_at = jnp.sum(jnp.where(ge, rc, 0)) - \
                jnp.sum(jnp.where(ge2, rc, 0))
            return h, cab, cnt_at

        def walk_big(target):
            z = jnp.int32(0)

            def cond(st):
                i, tot, found, sbase, stot = st
                return jnp.logical_and(i < _H // _L, found == 0)

            def body(st):
                i, tot, found, sbase, stot = st
                base = (_H // _L - 1 - i) * _L
                cvec = hist[pl.ds(base, _L)]
                csum = jnp.sum(cvec)
                hit = tot + csum >= target
                sbase = jnp.where(hit, base, sbase)
                stot = jnp.where(hit, tot, stot)
                found = jnp.where(hit, 1, found)
                return (i + 1, tot + csum, found, sbase, stot)

            _, _, _, sbase, stot = lax.while_loop(cond, body,
                                                  (z, z, z, z, z))
            sv = hist[pl.ds(sbase, _L)]
            return analyze(sv, sbase, stot, target)

        def walk_small(nchunks, target):
            z = jnp.int32(0)

            def step(base, carry):
                tot, found, sv, sbase, stot = carry
                cvec = hist[pl.ds(base, _L)]
                hist[pl.ds(base, _L)] = zi16
                csum = jnp.sum(cvec)
                hit = jnp.logical_and(found == 0, tot + csum >= target)
                sv = jnp.where(hit, cvec, sv)
                sbase = jnp.where(hit, base, sbase)
                stot = jnp.where(hit, tot, stot)
                found = jnp.where(hit, 1, found)
                return (tot + csum, found, sv, sbase, stot)

            un = 4 if nchunks % 4 == 0 else 1

            def body(i, carry):
                for u in range(un):
                    carry = step((nchunks - 1 - (i * un + u)) * _L, carry)
                return carry

            carry = lax.fori_loop(0, nchunks // un, body,
                                  (z, z, zi16, z, z))
            _, _, sv, sbase, stot = carry
            return analyze(sv, sbase, stot, target)

        h_in = h_in0
        h_out = [None, None]
        for j in range(_RPW):
            row = row0 + j
            xv = xv2[j % 2]
            outv = outv2[j % 2]
            kix = kix2[j % 2]
            h_in.wait()
            if j + 1 < _RPW:
                h_in = pltpu.async_copy(x_hbm.at[row + 1],
                                        xv2[(j + 1) % 2], si2[(j + 1) % 2])
            if j >= 2:
                h_out[j % 2].wait()

                @plsc.parallel_loop(0, _K // _L, unroll=4)
                def _unemit(i, kix=kix, outv=outv):
                    ix = kix[pl.ds(i * _L, _L)]
                    plsc.store_scatter(outv, [ix], zf16)

            @plsc.parallel_loop(0, _CHUNKS, unroll=8)
            def _p1(i, xv=xv):
                xb = xv[pl.ds(i * _L, _L)]
                sk = sortkey(lax.bitcast_convert_type(xb, jnp.int32))
                plsc.addupdate_scatter(hist, [(sk >> 20) + 2048], ones16)

            h1, cab1, _ = walk_big(jnp.int32(_K))
            pfx1 = h1 - 2048
            r1 = _K - cab1

            @plsc.parallel_loop(0, _H // _L, unroll=8)
            def _rezero(i):
                hist[pl.ds(i * _L, _L)] = zi16

            @plsc.parallel_loop(0, _CHUNKS, unroll=4, carry=jnp.int32(0))
            def pk(i, off, xv=xv):
                xb = xv[pl.ds(i * _L, _L)]
                sk = sortkey(lax.bitcast_convert_type(xb, jnp.int32))
                m = (sk >> 20) >= pfx1
                plsc.store_compressed(csk.at[pl.ds(off, _L)], sk, mask=m)
                plsc.store_compressed(cix.at[pl.ds(off, _L)],
                                      lanes + i * _L, mask=m)
                return off + jnp.max(plsc.all_reduce_population_count(m))

            cnt = pk
            nch = (cnt + _L - 1) // _L

            def refine(pshift, pfx, bshift, bmask, nbins, target):
                @plsc.parallel_loop(0, nch)
                def _rf(i):
                    base = i * _L
                    sk = csk[pl.ds(base, _L)]
                    m = jnp.logical_and(lanes < (cnt - base),
                                        (sk >> pshift) == pfx)
                    plsc.addupdate_scatter(hist, [(sk >> bshift) & bmask],
                                           ones16, mask=m)

                return walk_small(nbins // _L, target)

            b2a, cabA, _ = refine(20, pfx1, 12, 0xFF, 256, r1)
            pfx_a = (pfx1 << 8) | b2a
            r2a = r1 - cabA
            b2b, cabB, _ = refine(12, pfx_a, 4, 0xFF, 256, r2a)
            pfx_b = (pfx_a << 8) | b2b
            r2b = r2a - cabB
            b3, cabC, cnt_at = refine(4, pfx_b, 0, 0xF, 16, r2b)
            t = (pfx_b << 4) | b3
            tie_budget = r2b - cabC

            @plsc.parallel_loop(0, nch, carry=(jnp.int32(0), jnp.int32(0)))
            def sel(i, c, kix=kix):
                off, used = c
                base = i * _L
                sk = csk[pl.ds(base, _L)]
                ix = cix[pl.ds(base, _L)]
                valid = lanes < (cnt - base)
                gt = jnp.logical_and(valid, sk > t)
                eq = jnp.logical_and(valid, sk == t)
                eq_i = jnp.where(eq, 1, 0)
                tie_rank = used + plsc.cumsum(eq_i)
                keep = gt | jnp.logical_and(eq, tie_rank <= tie_budget)
                plsc.store_compressed(ksk.at[pl.ds(off, _L)], sk, mask=keep)
                plsc.store_compressed(kix.at[pl.ds(off, _L)], ix, mask=keep)
                off = off + jnp.max(plsc.all_reduce_population_count(keep))
                return (off, jnp.max(tie_rank))

            @plsc.parallel_loop(0, _K // _L, unroll=4)
            def _emit(i, kix=kix, outv=outv):
                sk = ksk[pl.ds(i * _L, _L)]
                ix = kix[pl.ds(i * _L, _L)]
                vals = lax.bitcast_convert_type(sortkey(sk), jnp.float32)
                plsc.store_scatter(outv, [ix], vals)

            h_out[j % 2] = pltpu.async_copy(outv, out_hbm.at[row],
                                            so2[j % 2])

        h_out[0].wait()
        h_out[1].wait()

    return wta(x)


# speedup vs baseline: 1.0616x; 1.0616x over previous
import functools

import jax
import jax.numpy as jnp
from jax import lax
from jax.experimental import pallas as pl
from jax.experimental.pallas import tpu as pltpu
from jax.experimental.pallas import tpu_sc as plsc

_K = 256
_ROWS = 128
_COLS = 8192
_L = 16
_CHUNKS = _COLS // _L
_NC = 2
_NS = 16
_NW = _NC * _NS
_RPW = _ROWS // _NW
_H = 4096


def kernel(x):
    mesh = plsc.VectorSubcoreMesh(core_axis_name="c", subcore_axis_name="s")

    @functools.partial(
        pl.kernel,
        mesh=mesh,
        out_type=jax.ShapeDtypeStruct((_ROWS, _COLS), jnp.float32),
        scratch_types=[
            pltpu.VMEM((_COLS,), jnp.float32),
            pltpu.VMEM((_COLS,), jnp.float32),
            pltpu.VMEM((_H,), jnp.int32),
            pltpu.VMEM((_COLS + _L,), jnp.int32),
            pltpu.VMEM((_COLS + _L,), jnp.int32),
            pltpu.VMEM((_COLS,), jnp.float32),
            pltpu.VMEM((_COLS,), jnp.float32),
            pltpu.VMEM((_K + _L,), jnp.int32),
            pltpu.VMEM((_K + _L,), jnp.int32),
            pltpu.VMEM((_K + _L,), jnp.int32),
            pltpu.SemaphoreType.DMA,
            pltpu.SemaphoreType.DMA,
            pltpu.SemaphoreType.DMA,
            pltpu.SemaphoreType.DMA,
        ],
        compiler_params=pltpu.CompilerParams(needs_layout_passes=False),
    )
    def wta(x_hbm, out_hbm, xva, xvb, hist, csk, cix, outva, outvb,
            ksk, kixa, kixb, sia, sib, soa, sob):
        zi16 = jnp.zeros((_L,), jnp.int32)
        zf16 = jnp.zeros((_L,), jnp.float32)
        ones16 = jnp.ones((_L,), jnp.int32)
        lanes = lax.iota(jnp.int32, _L)
        wid = lax.axis_index("s") * _NC + lax.axis_index("c")
        row0 = wid * _RPW
        xv2 = (xva, xvb)
        outv2 = (outva, outvb)
        kix2 = (kixa, kixb)
        si2 = (sia, sib)
        so2 = (soa, sob)

        h_in0 = pltpu.async_copy(x_hbm.at[row0], xva, sia)

        @plsc.parallel_loop(0, _H // _L, unroll=8)
        def _zero_hist(i):
            hist[pl.ds(i * _L, _L)] = zi16

        @plsc.parallel_loop(0, _CHUNKS, unroll=8)
        def _zero_out(i):
            outva[pl.ds(i * _L, _L)] = zf16
            outvb[pl.ds(i * _L, _L)] = zf16

        def sortkey(bits):
            return bits ^ ((bits >> 31) & jnp.int32(0x7FFFFFFF))

        def analyze(sv, sbase, stot, target):
            rc = lax.rev(sv, (0,))
            cum = stot + plsc.cumsum(rc)
            ge = cum >= target
            ge2 = (cum - rc) >= target
            popc = jnp.max(plsc.all_reduce_population_count(ge))
            h = sbase + popc - 1
            cab = stot + jnp.sum(jnp.where(ge, 0, rc))
            cnt_at = jnp.sum(jnp.where(ge, rc, 0)) - \
                jnp.sum(jnp.where(ge2, rc, 0))
            return h, cab, cnt_at

        def walk_big(target):
            z = jnp.int32(0)

            def cond(st):
                i, tot, found, sbase, stot = st
                return jnp.logical_and(i < _H // _L, found == 0)

            def body(st):
                i, tot, found, sbase, stot = st
                base = (_H // _L - 1 - i) * _L
                cvec = hist[pl.ds(base, _L)]
                csum = jnp.sum(cvec)
                hit = tot + csum >= target
                sbase = jnp.where(hit, base, sbase)
                stot = jnp.where(hit, tot, stot)
                found = jnp.where(hit, 1, found)
                return (i + 1, tot + csum, found, sbase, stot)

            _, _, _, sbase, stot = lax.while_loop(cond, body,
                                                  (z, z, z, z, z))
            sv = hist[pl.ds(sbase, _L)]
            return analyze(sv, sbase, stot, target)

        def walk_small(nchunks, target):
            z = jnp.int32(0)

            def step(base, carry):
                tot, found, sv, sbase, stot = carry
                cvec = hist[pl.ds(base, _L)]
                hist[pl.ds(base, _L)] = zi16
                csum = jnp.sum(cvec)
                hit = jnp.logical_and(found == 0, tot + csum >= target)
                sv = jnp.where(hit, cvec, sv)
                sbase = jnp.where(hit, base, sbase)
                stot = jnp.where(hit, tot, stot)
                found = jnp.where(hit, 1, found)
                return (tot + csum, found, sv, sbase, stot)

            un = 4 if nchunks % 4 == 0 else 1

            def body(i, carry):
                for u in range(un):
                    carry = step((nchunks - 1 - (i * un + u)) * _L, carry)
                return carry

            carry = lax.fori_loop(0, nchunks // un, body,
                                  (z, z, zi16, z, z))
            _, _, sv, sbase, stot = carry
            return analyze(sv, sbase, stot, target)

        h_in = h_in0
        h_out = [None, None]
        for j in range(_RPW):
            row = row0 + j
            xv = xv2[j % 2]
            outv = outv2[j % 2]
            kix = kix2[j % 2]
            h_in.wait()
            if j + 1 < _RPW:
                h_in = pltpu.async_copy(x_hbm.at[row + 1],
                                        xv2[(j + 1) % 2], si2[(j + 1) % 2])
            if j >= 2:
                h_out[j % 2].wait()

                @plsc.parallel_loop(0, _K // _L, unroll=4)
                def _unemit(i, kix=kix, outv=outv):
                    ix = kix[pl.ds(i * _L, _L)]
                    plsc.store_scatter(outv, [ix], zf16)

            @plsc.parallel_loop(0, _CHUNKS, unroll=8)
            def _p1(i, xv=xv):
                xb = xv[pl.ds(i * _L, _L)]
                sk = sortkey(lax.bitcast_convert_type(xb, jnp.int32))
                plsc.addupdate_scatter(hist, [(sk >> 20) + 2048], ones16)

            h1, cab1, _ = walk_small(_H // _L, jnp.int32(_K))
            pfx1 = h1 - 2048
            r1 = _K - cab1

            @plsc.parallel_loop(0, _CHUNKS, unroll=4, carry=jnp.int32(0))
            def pk(i, off, xv=xv):
                xb = xv[pl.ds(i * _L, _L)]
                sk = sortkey(lax.bitcast_convert_type(xb, jnp.int32))
                m = (sk >> 20) >= pfx1
                plsc.store_compressed(csk.at[pl.ds(off, _L)], sk, mask=m)
                plsc.store_compressed(cix.at[pl.ds(off, _L)],
                                      lanes + i * _L, mask=m)
                return off + jnp.max(plsc.all_reduce_population_count(m))

            cnt = pk
            nch = (cnt + _L - 1) // _L

            def refine(pshift, pfx, bshift, bmask, nbins, target):
                @plsc.parallel_loop(0, nch)
                def _rf(i):
                    base = i * _L
                    sk = csk[pl.ds(base, _L)]
                    m = jnp.logical_and(lanes < (cnt - base),
                                        (sk >> pshift) == pfx)
                    plsc.addupdate_scatter(hist, [(sk >> bshift) & bmask],
                                           ones16, mask=m)

                return walk_small(nbins // _L, target)

            b2a, cabA, _ = refine(20, pfx1, 12, 0xFF, 256, r1)
            pfx_a = (pfx1 << 8) | b2a
            r2a = r1 - cabA
            b2b, cabB, _ = refine(12, pfx_a, 4, 0xFF, 256, r2a)
            pfx_b = (pfx_a << 8) | b2b
            r2b = r2a - cabB
            b3, cabC, cnt_at = refine(4, pfx_b, 0, 0xF, 16, r2b)
            t = (pfx_b << 4) | b3
            tie_budget = r2b - cabC

            @plsc.parallel_loop(0, nch, carry=(jnp.int32(0), jnp.int32(0)))
            def sel(i, c, kix=kix):
                off, used = c
                base = i * _L
                sk = csk[pl.ds(base, _L)]
                ix = cix[pl.ds(base, _L)]
                valid = lanes < (cnt - base)
                gt = jnp.logical_and(valid, sk > t)
                eq = jnp.logical_and(valid, sk == t)
                eq_i = jnp.where(eq, 1, 0)
                tie_rank = used + plsc.cumsum(eq_i)
                keep = gt | jnp.logical_and(eq, tie_rank <= tie_budget)
                plsc.store_compressed(ksk.at[pl.ds(off, _L)], sk, mask=keep)
                plsc.store_compressed(kix.at[pl.ds(off, _L)], ix, mask=keep)
                off = off + jnp.max(plsc.all_reduce_population_count(keep))
                return (off, jnp.max(tie_rank))

            @plsc.parallel_loop(0, _K // _L, unroll=4)
            def _emit(i, kix=kix, outv=outv):
                sk = ksk[pl.ds(i * _L, _L)]
                ix = kix[pl.ds(i * _L, _L)]
                vals = lax.bitcast_convert_type(sortkey(sk), jnp.float32)
                plsc.store_scatter(outv, [ix], vals)

            h_out[j % 2] = pltpu.async_copy(outv, out_hbm.at[row],
                                            so2[j % 2])

        h_out[0].wait()
        h_out[1].wait()

    return wta(x)


# speedup vs baseline: 1.0757x; 1.0133x over previous
import functools

import jax
import jax.numpy as jnp
from jax import lax
from jax.experimental import pallas as pl
from jax.experimental.pallas import tpu as pltpu
from jax.experimental.pallas import tpu_sc as plsc

_K = 256
_ROWS = 128
_COLS = 8192
_L = 16
_CHUNKS = _COLS // _L
_NC = 2
_NS = 16
_NW = _NC * _NS
_RPW = _ROWS // _NW
_H = 4096


def kernel(x):
    mesh = plsc.VectorSubcoreMesh(core_axis_name="c", subcore_axis_name="s")

    @functools.partial(
        pl.kernel,
        mesh=mesh,
        out_type=jax.ShapeDtypeStruct((_ROWS, _COLS), jnp.float32),
        scratch_types=[
            pltpu.VMEM((_COLS,), jnp.float32),
            pltpu.VMEM((_H,), jnp.int32),
            pltpu.VMEM((_COLS + _L,), jnp.int32),
            pltpu.VMEM((_COLS + _L,), jnp.int32),
            pltpu.VMEM((_COLS,), jnp.float32),
            pltpu.VMEM((_K + _L,), jnp.int32),
            pltpu.VMEM((_K + _L,), jnp.int32),
        ],
        compiler_params=pltpu.CompilerParams(needs_layout_passes=False),
    )
    def wta(x_hbm, out_hbm, xv, hist, csk, cix, outv, ksk, kix):
        zi16 = jnp.zeros((_L,), jnp.int32)
        zf16 = jnp.zeros((_L,), jnp.float32)
        ones16 = jnp.ones((_L,), jnp.int32)
        lanes = lax.iota(jnp.int32, _L)
        wid = lax.axis_index("s") * _NC + lax.axis_index("c")

        @plsc.parallel_loop(0, _H // _L, unroll=8)
        def _zero_init(i):
            hist[pl.ds(i * _L, _L)] = zi16
            outv[pl.ds(i * _L, _L)] = zf16
            outv[pl.ds((i + _H // _L) * _L, _L)] = zf16

        def sortkey(bits):
            return bits ^ ((bits >> 31) & jnp.int32(0x7FFFFFFF))

        def walk(nchunks, target):
            z = jnp.int32(0)

            def step(base, carry):
                tot, found, sv, sbase, stot = carry
                cvec = hist[pl.ds(base, _L)]
                hist[pl.ds(base, _L)] = zi16
                csum = jnp.sum(cvec)
                hit = jnp.logical_and(found == 0, tot + csum >= target)
                sv = jnp.where(hit, cvec, sv)
                sbase = jnp.where(hit, base, sbase)
                stot = jnp.where(hit, tot, stot)
                found = jnp.where(hit, 1, found)
                return (tot + csum, found, sv, sbase, stot)

            un = 4 if nchunks % 4 == 0 else 1

            def body(i, carry):
                for u in range(un):
                    carry = step((nchunks - 1 - (i * un + u)) * _L, carry)
                return carry

            carry = lax.fori_loop(0, nchunks // un, body,
                                  (z, z, zi16, z, z))
            _, _, sv, sbase, stot = carry

            rc = lax.rev(sv, (0,))
            cum = stot + plsc.cumsum(rc)
            ge = cum >= target
            ge2 = (cum - rc) >= target
            popc = jnp.max(plsc.all_reduce_population_count(ge))
            h = sbase + popc - 1
            cab = stot + jnp.sum(jnp.where(ge, 0, rc))
            cnt_at = jnp.sum(jnp.where(ge, rc, 0)) - \
                jnp.sum(jnp.where(ge2, rc, 0))
            return h, cab, cnt_at

        def do_row(j, carry):
            row = wid * _RPW + j
            pltpu.sync_copy(x_hbm.at[row], xv)

            @plsc.parallel_loop(0, _CHUNKS, unroll=8)
            def _p1(i):
                xb = xv[pl.ds(i * _L, _L)]
                sk = sortkey(lax.bitcast_convert_type(xb, jnp.int32))
                plsc.addupdate_scatter(hist, [(sk >> 20) + 2048], ones16)

            h1, cab1, _ = walk(_H // _L, jnp.int32(_K))
            pfx1 = h1 - 2048
            r1 = _K - cab1

            @plsc.parallel_loop(0, _CHUNKS, unroll=4, carry=jnp.int32(0))
            def pk(i, off):
                xb = xv[pl.ds(i * _L, _L)]
                sk = sortkey(lax.bitcast_convert_type(xb, jnp.int32))
                m = (sk >> 20) >= pfx1
                plsc.store_compressed(csk.at[pl.ds(off, _L)], sk, mask=m)
                plsc.store_compressed(cix.at[pl.ds(off, _L)],
                                      lanes + i * _L, mask=m)
                return off + jnp.max(plsc.all_reduce_population_count(m))

            cnt = pk
            nch = (cnt + _L - 1) // _L

            def refine(pshift, pfx, bshift, bmask, nbins, target):
                @plsc.parallel_loop(0, nch)
                def _rf(i):
                    base = i * _L
                    sk = csk[pl.ds(base, _L)]
                    m = jnp.logical_and(lanes < (cnt - base),
                                        (sk >> pshift) == pfx)
                    plsc.addupdate_scatter(hist, [(sk >> bshift) & bmask],
                                           ones16, mask=m)

                return walk(nbins // _L, target)

            b2a, cabA, _ = refine(20, pfx1, 12, 0xFF, 256, r1)
            pfx_a = (pfx1 << 8) | b2a
            r2a = r1 - cabA
            b2b, cabB, _ = refine(12, pfx_a, 4, 0xFF, 256, r2a)
            pfx_b = (pfx_a << 8) | b2b
            r2b = r2a - cabB
            b3, cabC, cnt_at = refine(4, pfx_b, 0, 0xF, 16, r2b)
            t = (pfx_b << 4) | b3
            tie_budget = r2b - cabC

            @plsc.parallel_loop(0, nch, carry=(jnp.int32(0), jnp.int32(0)))
            def sel(i, c):
                off, used = c
                base = i * _L
                sk = csk[pl.ds(base, _L)]
                ix = cix[pl.ds(base, _L)]
                valid = lanes < (cnt - base)
                gt = jnp.logical_and(valid, sk > t)
                eq = jnp.logical_and(valid, sk == t)
                eq_i = jnp.where(eq, 1, 0)
                tie_rank = used + plsc.cumsum(eq_i)
                keep = gt | jnp.logical_and(eq, tie_rank <= tie_budget)
                plsc.store_compressed(ksk.at[pl.ds(off, _L)], sk, mask=keep)
                plsc.store_compressed(kix.at[pl.ds(off, _L)], ix, mask=keep)
                off = off + jnp.max(plsc.all_reduce_population_count(keep))
                return (off, jnp.max(tie_rank))

            @plsc.parallel_loop(0, _K // _L, unroll=4)
            def _emit(i):
                sk = ksk[pl.ds(i * _L, _L)]
                ix = kix[pl.ds(i * _L, _L)]
                vals = lax.bitcast_convert_type(sortkey(sk), jnp.float32)
                plsc.store_scatter(outv, [ix], vals)

            pltpu.sync_copy(outv, out_hbm.at[row])

            @plsc.parallel_loop(0, _K // _L, unroll=4)
            def _unemit(i):
                ix = kix[pl.ds(i * _L, _L)]
                plsc.store_scatter(outv, [ix], zf16)

            return carry

        lax.fori_loop(0, _RPW, do_row, 0)

    return wta(x)
